# baseline (speedup 1.0000x reference)
import jax
import jax.numpy as jnp
from jax.experimental import pallas as pl
from jax.experimental.pallas import tpu as pltpu

_GRID_SHAPE = (6, 64)
_BATCH = 32
_DIM = 512
_NUM_LAYERS = _GRID_SHAPE[0]
_SPATIAL = _GRID_SHAPE[1]
_NUM_CELLS = _NUM_LAYERS * _SPATIAL
_NUM_DIAGS = _NUM_LAYERS + _SPATIAL - 1


def _diag_body(x_ref, w_ref, b_ref, out0_ref, out1_ref,
               g0_ref, g1_ref, sem_ref):
    t = pl.program_id(0)
    cur = jax.lax.rem(t, 2)
    prev = 1 - cur

    @pl.when(t == 0)
    def _init():
        g0_ref[...] = jnp.zeros_like(g0_ref)
        g1_ref[...] = jnp.zeros_like(g1_ref)

    def valid(tt, l):
        return (tt >= l) & (tt <= l + _SPATIAL - 1)

    def copies(slot, tt, l):
        row = l * _SPATIAL + (tt - l)
        return (
            pltpu.make_async_copy(g0_ref.at[slot, l], out0_ref.at[row],
                                  sem_ref.at[slot, l, 0]),
            pltpu.make_async_copy(g1_ref.at[slot, l], out1_ref.at[row],
                                  sem_ref.at[slot, l, 1]),
        )

    for l in range(_NUM_LAYERS):
        @pl.when((t >= 2) & valid(t - 2, l))
        def _wait(l=l):
            c0, c1 = copies(cur, t - 2, l)
            c0.wait()
            c1.wait()

    s_x = jnp.minimum(t, _SPATIAL - 1)
    for l in range(_NUM_LAYERS):
        @pl.when(valid(t, l))
        def _cell(l=l):
            if l == 0:
                d0 = x_ref[s_x]
            else:
                d0 = g0_ref[prev, l - 1]
            d1 = g1_ref[prev, l]
            g0 = jnp.tanh(b_ref[l, 0] + d0 * w_ref[l, 0, 0] + d1 * w_ref[l, 0, 1])
            g1 = jnp.tanh(b_ref[l, 1] + d0 * w_ref[l, 1, 0] + d1 * w_ref[l, 1, 1])
            g0_ref[cur, l] = g0
            g1_ref[cur, l] = g1
            c0, c1 = copies(cur, t, l)
            c0.start()
            c1.start()

    @pl.when(t == _NUM_DIAGS - 1)
    def _drain():
        for l in range(_NUM_LAYERS):
            @pl.when(valid(t - 1, l))
            def _w1(l=l):
                c0, c1 = copies(prev, t - 1, l)
                c0.wait()
                c1.wait()

            @pl.when(valid(t, l))
            def _w2(l=l):
                c0, c1 = copies(cur, t, l)
                c0.wait()
                c1.wait()


def kernel(x, w, b):
    x_t = jnp.transpose(x, (1, 0, 2))
    out0, out1 = pl.pallas_call(
        _diag_body,
        grid=(_NUM_DIAGS,),
        in_specs=[
            pl.BlockSpec((_SPATIAL, _BATCH, _DIM), lambda t: (0, 0, 0)),
            pl.BlockSpec(w.shape, lambda t: (0, 0, 0, 0)),
            pl.BlockSpec(b.shape, lambda t: (0, 0, 0)),
        ],
        out_specs=[
            pl.BlockSpec(memory_space=pl.ANY),
            pl.BlockSpec(memory_space=pl.ANY),
        ],
        out_shape=[
            jax.ShapeDtypeStruct((_NUM_CELLS, _BATCH, _DIM), x.dtype),
            jax.ShapeDtypeStruct((_NUM_CELLS, _BATCH, _DIM), x.dtype),
        ],
        scratch_shapes=[
            pltpu.VMEM((2, _NUM_LAYERS, _BATCH, _DIM), x.dtype),
            pltpu.VMEM((2, _NUM_LAYERS, _BATCH, _DIM), x.dtype),
            pltpu.SemaphoreType.DMA((2, _NUM_LAYERS, 2)),
        ],
        compiler_params=pltpu.CompilerParams(
            dimension_semantics=("arbitrary",),
        ),
    )(x_t, w, b)
    return (out0, out1)


# speedup vs baseline: 1.0403x; 1.0403x over previous
import jax
import jax.numpy as jnp
from jax.experimental import pallas as pl
from jax.experimental.pallas import tpu as pltpu

_GRID_SHAPE = (6, 64)
_BATCH = 32
_DIM = 512
_NUM_LAYERS = _GRID_SHAPE[0]
_SPATIAL = _GRID_SHAPE[1]
_NUM_CELLS = _NUM_LAYERS * _SPATIAL
_NUM_DIAGS = _NUM_LAYERS + _SPATIAL - 1
_NUM_STEPS = (_NUM_DIAGS + 1) // 2


def _diag_body(x_ref, w_ref, b_ref, out0_ref, out1_ref,
               a0_ref, a1_ref, b0_ref, b1_ref, semA_ref, semB_ref):
    k = pl.program_id(0)
    t0 = 2 * k
    t1 = 2 * k + 1

    @pl.when(k == 0)
    def _init():
        a0_ref[...] = jnp.zeros_like(a0_ref)
        a1_ref[...] = jnp.zeros_like(a1_ref)
        b0_ref[...] = jnp.zeros_like(b0_ref)
        b1_ref[...] = jnp.zeros_like(b1_ref)

    def valid(tt, l):
        return (tt >= l) & (tt <= l + _SPATIAL - 1)

    def copies(s0, s1, sem, tt, l):
        row = l * _SPATIAL + (tt - l)
        return (
            pltpu.make_async_copy(s0.at[l], out0_ref.at[row], sem.at[l, 0]),
            pltpu.make_async_copy(s1.at[l], out1_ref.at[row], sem.at[l, 1]),
        )

    def half_step(tt, dst0, dst1, src0, src1, sem):
        for l in range(_NUM_LAYERS):
            @pl.when(valid(tt - 2, l))
            def _wait(l=l):
                c0, c1 = copies(dst0, dst1, sem, tt - 2, l)
                c0.wait()
                c1.wait()

        s_x = jnp.minimum(tt, _SPATIAL - 1)
        for l in range(_NUM_LAYERS):
            @pl.when(valid(tt, l))
            def _cell(l=l):
                d0 = x_ref[s_x] if l == 0 else src0[l - 1]
                d1 = src1[l]
                g0 = jnp.tanh(b_ref[l, 0] + d0 * w_ref[l, 0, 0]
                              + d1 * w_ref[l, 0, 1])
                g1 = jnp.tanh(b_ref[l, 1] + d0 * w_ref[l, 1, 0]
                              + d1 * w_ref[l, 1, 1])
                dst0[l] = g0
                dst1[l] = g1
                c0, c1 = copies(dst0, dst1, sem, tt, l)
                c0.start()
                c1.start()

    half_step(t0, a0_ref, a1_ref, b0_ref, b1_ref, semA_ref)
    half_step(t1, b0_ref, b1_ref, a0_ref, a1_ref, semB_ref)

    @pl.when(k == _NUM_STEPS - 1)
    def _drain():
        for tt, d0_, d1_, sem in ((t0, a0_ref, a1_ref, semA_ref),
                                  (t1, b0_ref, b1_ref, semB_ref)):
            for l in range(_NUM_LAYERS):
                @pl.when(valid(tt, l))
                def _w(l=l, tt=tt, d0_=d0_, d1_=d1_, sem=sem):
                    c0, c1 = copies(d0_, d1_, sem, tt, l)
                    c0.wait()
                    c1.wait()


def kernel(x, w, b):
    x_t = jnp.transpose(x, (1, 0, 2))
    slab = lambda: pltpu.VMEM((_NUM_LAYERS, _BATCH, _DIM), x.dtype)
    out0, out1 = pl.pallas_call(
        _diag_body,
        grid=(_NUM_STEPS,),
        in_specs=[
            pl.BlockSpec((_SPATIAL, _BATCH, _DIM), lambda k: (0, 0, 0)),
            pl.BlockSpec(w.shape, lambda k: (0, 0, 0, 0)),
            pl.BlockSpec(b.shape, lambda k: (0, 0, 0)),
        ],
        out_specs=[
            pl.BlockSpec(memory_space=pl.ANY),
            pl.BlockSpec(memory_space=pl.ANY),
        ],
        out_shape=[
            jax.ShapeDtypeStruct((_NUM_CELLS, _BATCH, _DIM), x.dtype),
            jax.ShapeDtypeStruct((_NUM_CELLS, _BATCH, _DIM), x.dtype),
        ],
        scratch_shapes=[
            slab(), slab(), slab(), slab(),
            pltpu.SemaphoreType.DMA((_NUM_LAYERS, 2)),
            pltpu.SemaphoreType.DMA((_NUM_LAYERS, 2)),
        ],
        compiler_params=pltpu.CompilerParams(
            dimension_semantics=("arbitrary",),
        ),
    )(x_t, w, b)
    return (out0, out1)
